# Initial kernel scaffold; baseline (speedup 1.0000x reference)
import jax
import jax.numpy as jnp
from jax.experimental import pallas as pl


def kernel(state, result_table, letter_table, col_table, row_table):
    raise NotImplementedError("write your pallas kernel here")


# speedup vs baseline: 8.0301x; 8.0301x over previous
import functools

import jax
import jax.numpy as jnp
from jax import lax
from jax.experimental import pallas as pl
from jax.experimental.pallas import tpu as pltpu
from jax.experimental.pallas import tpu_sc as plsc

EMBED = 128
NPOS = 30
NRES = 4
NLET = 28
NFUSED = NPOS * NRES * NLET
LANES = 128


def _tc_prep_body(s0_ref, s1_ref, rowt_ref, colt_ref, rest_ref, lett_ref,
                  idx_ref, fused_ref):
    nrows = s0_ref.shape[0]
    i0 = lax.broadcasted_iota(jnp.int32, (nrows, LANES), 0)
    i1 = lax.broadcasted_iota(jnp.int32, (nrows, LANES), 1)
    p = (i0 * LANES + i1) % NPOS
    idx_ref[...] = p * (NRES * NLET) + s0_ref[...] * NLET + s1_ref[...]
    pos = rowt_ref[...][:, None, :] + colt_ref[...][None, :, :]
    pos = pos.reshape(NPOS, EMBED)
    f = (pos[:, None, None, :]
         + rest_ref[...][None, :, None, :]
         + lett_ref[...][None, None, :, :])
    fused_ref[...] = f


def _sc_gather(fused, idx, n_rows):
    info = plsc.get_sparse_core_info()
    nw = info.num_cores * info.num_subcores
    n_chunks = idx.shape[0]
    assert n_chunks % nw == 0
    iters = n_chunks // nw
    mesh = plsc.VectorSubcoreMesh(core_axis_name="c", subcore_axis_name="s")

    @functools.partial(
        pl.kernel, mesh=mesh,
        out_type=jax.ShapeDtypeStruct((n_rows, EMBED), jnp.float32),
        scratch_types=[
            pltpu.VMEM((LANES,), jnp.int32),
            pltpu.VMEM((LANES, EMBED), jnp.float32),
            pltpu.SemaphoreType.DMA,
        ],
    )
    def k(fused_hbm, idx_hbm, out_hbm, idx_v, rows_v, sem):
        wid = lax.axis_index("s") * info.num_cores + lax.axis_index("c")

        def body(i, carry):
            j = wid * iters + i
            pltpu.sync_copy(idx_hbm.at[j], idx_v)
            pltpu.async_copy(fused_hbm.at[idx_v], rows_v, sem).wait()
            pltpu.sync_copy(rows_v, out_hbm.at[pl.ds(j * LANES, LANES)])
            return carry

        lax.fori_loop(0, iters, body, 0)

    return k(fused, idx)


def kernel(state, result_table, letter_table, col_table, row_table):
    b = state.shape[0]
    n_rows = b * NPOS
    n_chunks = n_rows // LANES
    s0 = state[..., 0].reshape(n_chunks, LANES)
    s1 = state[..., 1].reshape(n_chunks, LANES)

    idx, fused = pl.pallas_call(
        _tc_prep_body,
        out_shape=[
            jax.ShapeDtypeStruct((n_chunks, LANES), jnp.int32),
            jax.ShapeDtypeStruct((NPOS, NRES, NLET, EMBED), jnp.float32),
        ],
    )(s0, s1, row_table, col_table, result_table, letter_table)

    out = _sc_gather(fused.reshape(NFUSED, EMBED), idx, n_rows)
    return out.reshape(b, 6, 5, EMBED)


# speedup vs baseline: 8.1634x; 1.0166x over previous
import functools

import jax
import jax.numpy as jnp
from jax import lax
from jax.experimental import pallas as pl
from jax.experimental.pallas import tpu as pltpu
from jax.experimental.pallas import tpu_sc as plsc

EMBED = 128
NPOS = 30
NRES = 4
NLET = 28
NFUSED = NPOS * NRES * NLET
LANES = 128


def _tc_prep_body(s0_ref, s1_ref, rowt_ref, colt_ref, rest_ref, lett_ref,
                  idx_ref, fused_ref):
    nrows = s0_ref.shape[0]
    i0 = lax.broadcasted_iota(jnp.int32, (nrows, LANES), 0)
    i1 = lax.broadcasted_iota(jnp.int32, (nrows, LANES), 1)
    p = (i0 * LANES + i1) % NPOS
    idx_ref[...] = p * (NRES * NLET) + s0_ref[...] * NLET + s1_ref[...]
    pos = rowt_ref[...][:, None, :] + colt_ref[...][None, :, :]
    pos = pos.reshape(NPOS, EMBED)
    f = (pos[:, None, None, :]
         + rest_ref[...][None, :, None, :]
         + lett_ref[...][None, None, :, :])
    fused_ref[...] = f


def _sc_gather(fused, idx, n_rows):
    info = plsc.get_sparse_core_info()
    nw = info.num_cores * info.num_subcores
    n_chunks = idx.shape[0]
    assert n_chunks % nw == 0
    iters = n_chunks // nw
    mesh = plsc.VectorSubcoreMesh(core_axis_name="c", subcore_axis_name="s")

    @functools.partial(
        pl.kernel, mesh=mesh,
        out_type=jax.ShapeDtypeStruct((n_rows, EMBED), jnp.float32),
        scratch_types=[
            pltpu.VMEM((LANES,), jnp.int32),
            pltpu.VMEM((LANES,), jnp.int32),
            pltpu.VMEM((LANES, EMBED), jnp.float32),
            pltpu.VMEM((LANES, EMBED), jnp.float32),
            pltpu.SemaphoreType.DMA,
            pltpu.SemaphoreType.DMA,
            pltpu.SemaphoreType.DMA,
            pltpu.SemaphoreType.DMA,
            pltpu.SemaphoreType.DMA,
            pltpu.SemaphoreType.DMA,
        ],
    )
    def k(fused_hbm, idx_hbm, out_hbm, idx_v0, idx_v1, rows_v0, rows_v1,
          si0, si1, sg0, sg1, ss0, ss1):
        wid = lax.axis_index("s") * info.num_cores + lax.axis_index("c")
        base = wid * iters
        idx_v = (idx_v0, idx_v1)
        rows_v = (rows_v0, rows_v1)
        si = (si0, si1)
        sg = (sg0, sg1)
        ss = (ss0, ss1)

        def idx_copy(i, b):
            return pltpu.make_async_copy(idx_hbm.at[base + i], idx_v[b], si[b])

        def gather(b):
            return pltpu.make_async_copy(fused_hbm.at[idx_v[b]], rows_v[b],
                                         sg[b])

        def scatter(i, b):
            dst = out_hbm.at[pl.ds((base + i) * LANES, LANES)]
            return pltpu.make_async_copy(rows_v[b], dst, ss[b])

        idx_copy(0, 0).start()
        idx_copy(1, 1).start()

        def body(it, carry):
            for b in (0, 1):
                i = it * 2 + b
                idx_copy(i, b).wait()

                @pl.when(it >= 1)
                def _():
                    scatter(i, b).wait()

                g = gather(b)
                g.start()
                g.wait()
                @pl.when(it < niter - 1)
                def _():
                    idx_copy(i + 2, b).start()

                scatter(i, b).start()
            return carry

        niter = iters // 2
        lax.fori_loop(0, niter, body, 0)
        scatter(0, 0).wait()
        scatter(0, 1).wait()

    return k(fused, idx)


def kernel(state, result_table, letter_table, col_table, row_table):
    b = state.shape[0]
    n_rows = b * NPOS
    n_chunks = n_rows // LANES
    s0 = state[..., 0].reshape(n_chunks, LANES)
    s1 = state[..., 1].reshape(n_chunks, LANES)

    idx, fused = pl.pallas_call(
        _tc_prep_body,
        out_shape=[
            jax.ShapeDtypeStruct((n_chunks, LANES), jnp.int32),
            jax.ShapeDtypeStruct((NPOS, NRES, NLET, EMBED), jnp.float32),
        ],
    )(s0, s1, row_table, col_table, result_table, letter_table)

    out = _sc_gather(fused.reshape(NFUSED, EMBED), idx, n_rows)
    return out.reshape(b, 6, 5, EMBED)


# speedup vs baseline: 11.1930x; 1.3711x over previous
import functools

import jax
import jax.numpy as jnp
from jax import lax
from jax.experimental import pallas as pl
from jax.experimental.pallas import tpu as pltpu
from jax.experimental.pallas import tpu_sc as plsc

EMBED = 128
NPOS = 30
NRES = 4
NLET = 28
NFUSED = NPOS * NRES * NLET
LANES = 128


def _tc_prep_body(s0_ref, s1_ref, rowt_ref, colt_ref, rest_ref, lett_ref,
                  idx_ref, fused_ref):
    nrows = s0_ref.shape[0]
    i0 = lax.broadcasted_iota(jnp.int32, (nrows, LANES), 0)
    i1 = lax.broadcasted_iota(jnp.int32, (nrows, LANES), 1)
    p = (i0 * LANES + i1) % NPOS
    idx_ref[...] = p * (NRES * NLET) + s0_ref[...] * NLET + s1_ref[...]
    pos = rowt_ref[...][:, None, :] + colt_ref[...][None, :, :]
    pos = pos.reshape(NPOS, EMBED)
    f = (pos[:, None, None, :]
         + rest_ref[...][None, :, None, :]
         + lett_ref[...][None, None, :, :])
    fused_ref[...] = f


def _sc_gather(fused, idx_flat, batch):
    info = plsc.get_sparse_core_info()
    nw = info.num_cores * info.num_subcores
    nb = 4
    chunk = nb * NPOS
    assert batch % (nw * nb) == 0
    iters = batch // (nw * nb)
    mesh = plsc.VectorSubcoreMesh(core_axis_name="c", subcore_axis_name="s")

    @functools.partial(
        pl.kernel, mesh=mesh,
        out_type=jax.ShapeDtypeStruct((batch, 6, 5, EMBED), jnp.float32),
        scratch_types=[
            pltpu.VMEM((chunk,), jnp.int32),
            pltpu.VMEM((chunk,), jnp.int32),
            pltpu.VMEM((chunk, EMBED), jnp.float32),
            pltpu.VMEM((chunk, EMBED), jnp.float32),
            pltpu.SemaphoreType.DMA,
            pltpu.SemaphoreType.DMA,
            pltpu.SemaphoreType.DMA,
            pltpu.SemaphoreType.DMA,
            pltpu.SemaphoreType.DMA,
            pltpu.SemaphoreType.DMA,
        ],
    )
    def k(fused_hbm, idx_hbm, out_hbm, idx_v0, idx_v1, rows_v0, rows_v1,
          si0, si1, sg0, sg1, ss0, ss1):
        wid = lax.axis_index("s") * info.num_cores + lax.axis_index("c")
        row0 = wid * iters * chunk
        b0 = wid * iters * nb
        idx_v = (idx_v0, idx_v1)
        rows_v = (rows_v0, rows_v1)
        si = (si0, si1)
        sg = (sg0, sg1)
        ss = (ss0, ss1)

        def idx_copy(i, b):
            src = idx_hbm.at[pl.ds(row0 + i * chunk, chunk)]
            return pltpu.make_async_copy(src, idx_v[b], si[b])

        def gather(b):
            return pltpu.make_async_copy(fused_hbm.at[idx_v[b]], rows_v[b],
                                         sg[b])

        def scatter_all(i, b):
            for bb in range(nb):
                for r in range(6):
                    src = rows_v[b].at[pl.ds(bb * NPOS + r * 5, 5)]
                    dst = out_hbm.at[b0 + i * nb + bb, r]
                    pltpu.async_copy(src, dst, ss[b])

        def scatter_drain(b):
            for bb in range(nb):
                for r in range(6):
                    src = rows_v[b].at[pl.ds(bb * NPOS + r * 5, 5)]
                    pltpu.make_async_copy(src, out_hbm.at[0, 0], ss[b]).wait()

        idx_copy(0, 0).start()
        idx_copy(1, 1).start()

        def body(it, carry):
            for b in (0, 1):
                i = it * 2 + b
                idx_copy(i, b).wait()

                @pl.when(it >= 1)
                def _():
                    scatter_drain(b)

                g = gather(b)
                g.start()
                g.wait()
                @pl.when(it < niter - 1)
                def _():
                    idx_copy(i + 2, b).start()

                scatter_all(i, b)
            return carry

        niter = iters // 2
        lax.fori_loop(0, niter, body, 0)
        scatter_drain(0)
        scatter_drain(1)

    return k(fused, idx_flat)


def kernel(state, result_table, letter_table, col_table, row_table):
    b = state.shape[0]
    n_rows = b * NPOS
    n_chunks = n_rows // LANES
    s0 = state[..., 0].reshape(n_chunks, LANES)
    s1 = state[..., 1].reshape(n_chunks, LANES)

    idx, fused = pl.pallas_call(
        _tc_prep_body,
        out_shape=[
            jax.ShapeDtypeStruct((n_chunks, LANES), jnp.int32),
            jax.ShapeDtypeStruct((NPOS, NRES, NLET, EMBED), jnp.float32),
        ],
    )(s0, s1, row_table, col_table, result_table, letter_table)

    return _sc_gather(fused.reshape(NFUSED, EMBED), idx.reshape(n_rows), b)
